# Initial kernel scaffold; baseline (speedup 1.0000x reference)
import jax
import jax.numpy as jnp
from jax.experimental import pallas as pl


def kernel(x, edge_index, edge_weight, batch, W1, b1, W2, b2, W3, b3, W4, b4, fc1_w, fc1_b, fc2_w, fc2_b):
    raise NotImplementedError("write your pallas kernel here")


# speedup vs baseline: 2.0081x; 2.0081x over previous
import functools

import jax
import jax.numpy as jnp
from jax import lax
from jax.experimental import pallas as pl
from jax.experimental.pallas import tpu as pltpu
from jax.experimental.pallas import tpu_sc as plsc

N = 10000
E = 320000
D = 128
H = 128
C = 10
G = 64

NC = 2
NS = 16
NT = NC * NS
BSN = 312
LAST = N - (NT - 1) * BSN
K = 6400
NCHUNK = E // K
BLK = 128
BUF = 144
QH = H // 16


def _mesh():
    return plsc.VectorSubcoreMesh(core_axis_name="c", subcore_axis_name="s",
                                  num_cores=NC, num_subcores=NS)


def _agg_kernel(row_hbm, col_hbm, w_hbm, table_hbm, out_hbm,
                rst, cst, wst, rbuf, cbuf, wbuf, rows_v, acc,
                off_sm, sem_r, sem_c, sem_w, sem_g):
    c = lax.axis_index("c")
    s = lax.axis_index("s")
    tid = c * NS + s
    lo = tid * BSN
    hi = jnp.where(tid == NT - 1, jnp.int32(N), lo + BSN)
    zi = jnp.zeros((16,), jnp.int32)
    zf = jnp.zeros((16,), jnp.float32)
    lanes = lax.iota(jnp.int32, 16)

    def za(r, _):
        for q in range(QH):
            acc[r, pl.ds(q * 16, 16)] = zf
        return 0
    lax.fori_loop(0, LAST, za, 0)
    for g in range(BUF // 16):
        rbuf[pl.ds(g * 16, 16)] = zi
        cbuf[pl.ds(g * 16, 16)] = zi
        wbuf[pl.ds(g * 16, 16)] = zf

    def process(ng):
        pltpu.async_copy(table_hbm.at[rbuf.at[pl.ds(0, BLK)]],
                         rows_v, sem_g).wait()

        def gbody(gb, _):
            @pl.when(gb < ng)
            def _():
                cl16 = cbuf[pl.ds(gb * 16, 16)]
                w16 = wbuf[pl.ds(gb * 16, 16)]
                for r in range(16):
                    cl = cl16[r]
                    ws = w16[r]
                    j = gb * 16 + r
                    for q in range(QH):
                        sl = pl.ds(q * 16, 16)
                        acc[cl, sl] = acc[cl, sl] + rows_v[j, sl] * ws
            return 0
        lax.fori_loop(0, BLK // 16, gbody, 0)

    def group(goff, _):
        row16 = rst[pl.ds(goff, 16)]
        col16 = cst[pl.ds(goff, 16)]
        w16 = wst[pl.ds(goff, 16)]
        m = (col16 >= lo) & (col16 < hi)
        mi = jnp.where(m, jnp.int32(1), jnp.int32(0))
        cs = mi
        for d in (1, 2, 4, 8):
            sh = cs.at[jnp.maximum(lanes - d, 0)].get(
                mode="promise_in_bounds")
            cs = cs + jnp.where(lanes >= d, sh, jnp.int32(0))
        pc = cs[15]

        @pl.when(pc > 0)
        def _():
            enc = jnp.where(m, cs - 1, jnp.int32(16))
            src = lanes
            for j in range(16):
                ej = enc[j]
                src = jnp.where(lanes == ej, jnp.int32(j), src)
            rapp = row16.at[src].get(mode="promise_in_bounds")
            capp = col16.at[src].get(mode="promise_in_bounds") - lo
            wapp = w16.at[src].get(mode="promise_in_bounds")
            off = off_sm[0]
            rbuf[pl.ds(off, 16)] = rapp
            cbuf[pl.ds(off, 16)] = capp
            wbuf[pl.ds(off, 16)] = wapp
            offn = off + pc

            @pl.when(offn >= BLK)
            def _():
                process(jnp.int32(BLK // 16))
                rbuf[pl.ds(0, 16)] = rbuf[pl.ds(BLK, 16)]
                cbuf[pl.ds(0, 16)] = cbuf[pl.ds(BLK, 16)]
                wbuf[pl.ds(0, 16)] = wbuf[pl.ds(BLK, 16)]
            off_sm[0] = jnp.where(offn >= BLK, offn - BLK, offn)
        return 0

    def chunk(i, _):
        base = i * K
        cp_r = pltpu.async_copy(row_hbm.at[pl.ds(base, K)], rst, sem_r)
        cp_c = pltpu.async_copy(col_hbm.at[pl.ds(base, K)], cst, sem_c)
        cp_w = pltpu.async_copy(w_hbm.at[pl.ds(base, K)], wst, sem_w)
        cp_r.wait()
        cp_c.wait()
        cp_w.wait()
        return lax.fori_loop(0, K // 16, lambda gi, o: group(gi * 16, o), 0)

    off_sm[0] = jnp.int32(0)
    lax.fori_loop(0, NCHUNK, chunk, 0)
    off = off_sm[0]
    cbuf[pl.ds(off, 16)] = zi
    wbuf[pl.ds(off, 16)] = zf
    process((off + 15) // 16)

    @pl.when(tid < NT - 1)
    def _():
        pltpu.sync_copy(acc.at[pl.ds(0, BSN)], out_hbm.at[pl.ds(lo, BSN)])

    @pl.when(tid == NT - 1)
    def _():
        pltpu.sync_copy(acc.at[pl.ds(0, LAST)], out_hbm.at[pl.ds(lo, LAST)])


@functools.lru_cache(maxsize=None)
def _built_agg():
    return pl.kernel(
        _agg_kernel,
        out_type=jax.ShapeDtypeStruct((N, H), jnp.float32),
        mesh=_mesh(),
        scratch_types=[
            pltpu.VMEM((K,), jnp.int32),
            pltpu.VMEM((K,), jnp.int32),
            pltpu.VMEM((K,), jnp.float32),
            pltpu.VMEM((BUF,), jnp.int32),
            pltpu.VMEM((BUF,), jnp.int32),
            pltpu.VMEM((BUF,), jnp.float32),
            pltpu.VMEM((BLK, H), jnp.float32),
            pltpu.VMEM((LAST, H), jnp.float32),
            pltpu.SMEM((8,), jnp.int32),
            pltpu.SemaphoreType.DMA,
            pltpu.SemaphoreType.DMA,
            pltpu.SemaphoreType.DMA,
            pltpu.SemaphoreType.DMA,
        ],
    )


def _agg_call(row, col, w, table):
    return _built_agg()(row, col, w, table)


def _elu(t):
    return jnp.where(t > 0, t, jnp.exp(jnp.minimum(t, 0.0)) - 1.0)


def _prep_body(x_ref, w1_ref, deg_ref, dinv_ref, tab_ref):
    deg = deg_ref[:, 0:1] + 1.0
    dinv = lax.rsqrt(deg)
    dinv_ref[...] = dinv
    tab_ref[...] = jnp.dot(x_ref[...], w1_ref[...],
                           preferred_element_type=jnp.float32) * dinv


def _prep_call(x, W1, deg):
    return pl.pallas_call(
        _prep_body,
        out_shape=(jax.ShapeDtypeStruct((N, 1), jnp.float32),
                   jax.ShapeDtypeStruct((N, H), jnp.float32)),
    )(x, W1, deg)


def _agg_dense(acc_ref, tab_ref, dinv_ref, b_ref):
    t = acc_ref[...] + tab_ref[...]
    return _elu(t * dinv_ref[...] + b_ref[...][None, :])


def _mid_body(acc_ref, tab_ref, dinv_ref, b_ref, wn_ref, out_ref):
    t = _agg_dense(acc_ref, tab_ref, dinv_ref, b_ref)
    out_ref[...] = jnp.dot(t, wn_ref[...],
                           preferred_element_type=jnp.float32) * dinv_ref[...]


def _mid_call(acc, tab, dinv, b, Wn):
    return pl.pallas_call(
        _mid_body,
        out_shape=jax.ShapeDtypeStruct((N, H), jnp.float32),
    )(acc, tab, dinv, b, Wn)


def _final_body(acc_ref, tab_ref, dinv_ref, b_ref, batch_ref,
                fc1w_ref, fc1b_ref, fc2w_ref, fc2b_ref, out_ref):
    t = _agg_dense(acc_ref, tab_ref, dinv_ref, b_ref)
    gids = lax.broadcasted_iota(jnp.int32, (G, N), 0)
    onehot = (gids == batch_ref[...]).astype(jnp.float32)
    sums = jnp.dot(onehot, t, preferred_element_type=jnp.float32)
    cnt = jnp.sum(onehot, axis=1, keepdims=True)
    pooled = sums / jnp.maximum(cnt, 1.0)
    z = jnp.dot(pooled, fc1w_ref[...],
                preferred_element_type=jnp.float32) + fc1b_ref[...][None, :]
    z = _elu(z)
    o = jnp.dot(z, fc2w_ref[...],
                preferred_element_type=jnp.float32) + fc2b_ref[...][None, :]
    m = jnp.max(o, axis=1, keepdims=True)
    lse = m + jnp.log(jnp.sum(jnp.exp(o - m), axis=1, keepdims=True))
    out_ref[...] = o - lse


def _final_call(acc, tab, dinv, b4, batch2d, fc1_w, fc1_b, fc2_w, fc2_b):
    return pl.pallas_call(
        _final_body,
        out_shape=jax.ShapeDtypeStruct((G, C), jnp.float32),
    )(acc, tab, dinv, b4, batch2d, fc1_w, fc1_b, fc2_w, fc2_b)


def kernel(x, edge_index, edge_weight, batch,
           W1, b1, W2, b2, W3, b3, W4, b4,
           fc1_w, fc1_b, fc2_w, fc2_b):
    row = edge_index[0]
    col = edge_index[1]
    batch2d = batch.reshape(1, N)

    deg = _agg_call(row, col, edge_weight, jnp.ones((N, H), jnp.float32))
    dinv, tab = _prep_call(x, W1, deg)
    for (bl, Wn) in ((b1, W2), (b2, W3), (b3, W4)):
        acc = _agg_call(row, col, edge_weight, tab)
        tab = _mid_call(acc, tab, dinv, bl, Wn)
    acc = _agg_call(row, col, edge_weight, tab)
    return _final_call(acc, tab, dinv, b4, batch2d,
                       fc1_w, fc1_b, fc2_w, fc2_b)


# speedup vs baseline: 2.1702x; 1.0807x over previous
import functools

import jax
import jax.numpy as jnp
from jax import lax
from jax.experimental import pallas as pl
from jax.experimental.pallas import tpu as pltpu
from jax.experimental.pallas import tpu_sc as plsc

N = 10000
E = 320000
D = 128
H = 128
C = 10
G = 64

NC = 2
NS = 16
NT = NC * NS
BSN = 312
LAST = N - (NT - 1) * BSN
K = 6400
NCHUNK = E // K
BLK = 128
BUF = 144
QH = H // 16


def _mesh():
    return plsc.VectorSubcoreMesh(core_axis_name="c", subcore_axis_name="s",
                                  num_cores=NC, num_subcores=NS)


def _agg_kernel(row_hbm, col_hbm, w_hbm, table_hbm, out_hbm,
                rst, cst, wst, rbuf, cbuf, wbuf, rows_v, acc,
                off_sm, sem_r, sem_c, sem_w, sem_g):
    c = lax.axis_index("c")
    s = lax.axis_index("s")
    tid = c * NS + s
    lo = tid * BSN
    hi = jnp.where(tid == NT - 1, jnp.int32(N), lo + BSN)
    zi = jnp.zeros((16,), jnp.int32)
    zf = jnp.zeros((16,), jnp.float32)
    lanes = lax.iota(jnp.int32, 16)

    def za(r, _):
        for q in range(QH):
            acc[r, pl.ds(q * 16, 16)] = zf
        return 0
    lax.fori_loop(0, LAST, za, 0)
    for g in range(BUF // 16):
        rbuf[pl.ds(g * 16, 16)] = zi
        cbuf[pl.ds(g * 16, 16)] = zi
        wbuf[pl.ds(g * 16, 16)] = zf

    def process(ng):
        pltpu.async_copy(table_hbm.at[rbuf.at[pl.ds(0, BLK)]],
                         rows_v, sem_g).wait()

        def gbody(gb, _):
            @pl.when(gb < ng)
            def _():
                cl16 = cbuf[pl.ds(gb * 16, 16)]
                w16 = wbuf[pl.ds(gb * 16, 16)]
                for r in range(16):
                    cl = cl16[r]
                    ws = w16[r]
                    j = gb * 16 + r
                    for q in range(QH):
                        sl = pl.ds(q * 16, 16)
                        acc[cl, sl] = acc[cl, sl] + rows_v[j, sl] * ws
            return 0
        lax.fori_loop(0, BLK // 16, gbody, 0)

    def group(goff, _):
        row16 = rst[pl.ds(goff, 16)]
        col16 = cst[pl.ds(goff, 16)]
        w16 = wst[pl.ds(goff, 16)]
        m = (col16 >= lo) & (col16 < hi)
        mi = jnp.where(m, jnp.int32(1), jnp.int32(0))
        cs = mi
        for d in (1, 2, 4, 8):
            sh = cs.at[jnp.maximum(lanes - d, 0)].get(
                mode="promise_in_bounds")
            cs = cs + jnp.where(lanes >= d, sh, jnp.int32(0))
        pc = cs[15]

        @pl.when(pc > 0)
        def _():
            enc = jnp.where(m, cs - 1, jnp.int32(16))
            src = lanes
            for j in range(16):
                ej = enc[j]
                src = jnp.where(lanes == ej, jnp.int32(j), src)
            rapp = row16.at[src].get(mode="promise_in_bounds")
            capp = col16.at[src].get(mode="promise_in_bounds") - lo
            wapp = w16.at[src].get(mode="promise_in_bounds")
            off = off_sm[0]
            rbuf[pl.ds(off, 16)] = rapp
            cbuf[pl.ds(off, 16)] = capp
            wbuf[pl.ds(off, 16)] = wapp
            offn = off + pc

            @pl.when(offn >= BLK)
            def _():
                process(jnp.int32(BLK // 16))
                rbuf[pl.ds(0, 16)] = rbuf[pl.ds(BLK, 16)]
                cbuf[pl.ds(0, 16)] = cbuf[pl.ds(BLK, 16)]
                wbuf[pl.ds(0, 16)] = wbuf[pl.ds(BLK, 16)]
            off_sm[0] = jnp.where(offn >= BLK, offn - BLK, offn)
        return 0

    def chunk(i, _):
        base = i * K
        cp_r = pltpu.async_copy(row_hbm.at[pl.ds(base, K)], rst, sem_r)
        cp_c = pltpu.async_copy(col_hbm.at[pl.ds(base, K)], cst, sem_c)
        cp_w = pltpu.async_copy(w_hbm.at[pl.ds(base, K)], wst, sem_w)
        cp_r.wait()
        cp_c.wait()
        cp_w.wait()
        return lax.fori_loop(0, K // 16, lambda gi, o: group(gi * 16, o), 0)

    off_sm[0] = jnp.int32(0)
    lax.fori_loop(0, NCHUNK, chunk, 0)
    off = off_sm[0]
    cbuf[pl.ds(off, 16)] = zi
    wbuf[pl.ds(off, 16)] = zf
    process((off + 15) // 16)

    @pl.when(tid < NT - 1)
    def _():
        pltpu.sync_copy(acc.at[pl.ds(0, BSN)], out_hbm.at[pl.ds(lo, BSN)])

    @pl.when(tid == NT - 1)
    def _():
        pltpu.sync_copy(acc.at[pl.ds(0, LAST)], out_hbm.at[pl.ds(lo, LAST)])


def _deg_kernel(col_hbm, w_hbm, out_hbm,
                cst, wst, cbuf, wbuf, acc, off_sm, sem_c, sem_w):
    c = lax.axis_index("c")
    s = lax.axis_index("s")
    tid = c * NS + s
    lo = tid * BSN
    hi = jnp.where(tid == NT - 1, jnp.int32(N), lo + BSN)
    zi = jnp.zeros((16,), jnp.int32)
    zf = jnp.zeros((16,), jnp.float32)
    lanes = lax.iota(jnp.int32, 16)

    def za(r, _):
        acc[r, pl.ds(0, 16)] = zf
        return 0
    lax.fori_loop(0, LAST, za, 0)
    for g in range(BUF // 16):
        cbuf[pl.ds(g * 16, 16)] = zi
        wbuf[pl.ds(g * 16, 16)] = zf

    def process(ng):
        def gbody(gb, _):
            @pl.when(gb < ng)
            def _():
                cl16 = cbuf[pl.ds(gb * 16, 16)]
                w16 = wbuf[pl.ds(gb * 16, 16)]
                for r in range(16):
                    cl = cl16[r]
                    acc[cl, pl.ds(0, 16)] = acc[cl, pl.ds(0, 16)] + w16[r]
            return 0
        lax.fori_loop(0, BLK // 16, gbody, 0)

    def group(goff, _):
        col16 = cst[pl.ds(goff, 16)]
        w16 = wst[pl.ds(goff, 16)]
        m = (col16 >= lo) & (col16 < hi)
        mi = jnp.where(m, jnp.int32(1), jnp.int32(0))
        cs = mi
        for d in (1, 2, 4, 8):
            sh = cs.at[jnp.maximum(lanes - d, 0)].get(
                mode="promise_in_bounds")
            cs = cs + jnp.where(lanes >= d, sh, jnp.int32(0))
        pc = cs[15]

        @pl.when(pc > 0)
        def _():
            enc = jnp.where(m, cs - 1, jnp.int32(16))
            src = lanes
            for j in range(16):
                ej = enc[j]
                src = jnp.where(lanes == ej, jnp.int32(j), src)
            capp = col16.at[src].get(mode="promise_in_bounds") - lo
            wapp = w16.at[src].get(mode="promise_in_bounds")
            off = off_sm[0]
            cbuf[pl.ds(off, 16)] = capp
            wbuf[pl.ds(off, 16)] = wapp
            offn = off + pc

            @pl.when(offn >= BLK)
            def _():
                process(jnp.int32(BLK // 16))
                cbuf[pl.ds(0, 16)] = cbuf[pl.ds(BLK, 16)]
                wbuf[pl.ds(0, 16)] = wbuf[pl.ds(BLK, 16)]
            off_sm[0] = jnp.where(offn >= BLK, offn - BLK, offn)
        return 0

    def chunk(i, _):
        base = i * K
        cp_c = pltpu.async_copy(col_hbm.at[pl.ds(base, K)], cst, sem_c)
        cp_w = pltpu.async_copy(w_hbm.at[pl.ds(base, K)], wst, sem_w)
        cp_c.wait()
        cp_w.wait()
        return lax.fori_loop(0, K // 16, lambda gi, o: group(gi * 16, o), 0)

    off_sm[0] = jnp.int32(0)
    lax.fori_loop(0, NCHUNK, chunk, 0)
    off = off_sm[0]
    cbuf[pl.ds(off, 16)] = zi
    wbuf[pl.ds(off, 16)] = zf
    process((off + 15) // 16)

    @pl.when(tid < NT - 1)
    def _():
        pltpu.sync_copy(acc.at[pl.ds(0, BSN)], out_hbm.at[pl.ds(lo, BSN)])

    @pl.when(tid == NT - 1)
    def _():
        pltpu.sync_copy(acc.at[pl.ds(0, LAST)], out_hbm.at[pl.ds(lo, LAST)])


@functools.lru_cache(maxsize=None)
def _built_deg():
    return pl.kernel(
        _deg_kernel,
        out_type=jax.ShapeDtypeStruct((N, 16), jnp.float32),
        mesh=_mesh(),
        scratch_types=[
            pltpu.VMEM((K,), jnp.int32),
            pltpu.VMEM((K,), jnp.float32),
            pltpu.VMEM((BUF,), jnp.int32),
            pltpu.VMEM((BUF,), jnp.float32),
            pltpu.VMEM((LAST, 16), jnp.float32),
            pltpu.SMEM((8,), jnp.int32),
            pltpu.SemaphoreType.DMA,
            pltpu.SemaphoreType.DMA,
        ],
    )


@functools.lru_cache(maxsize=None)
def _built_agg():
    return pl.kernel(
        _agg_kernel,
        out_type=jax.ShapeDtypeStruct((N, H), jnp.float32),
        mesh=_mesh(),
        scratch_types=[
            pltpu.VMEM((K,), jnp.int32),
            pltpu.VMEM((K,), jnp.int32),
            pltpu.VMEM((K,), jnp.float32),
            pltpu.VMEM((BUF,), jnp.int32),
            pltpu.VMEM((BUF,), jnp.int32),
            pltpu.VMEM((BUF,), jnp.float32),
            pltpu.VMEM((BLK, H), jnp.float32),
            pltpu.VMEM((LAST, H), jnp.float32),
            pltpu.SMEM((8,), jnp.int32),
            pltpu.SemaphoreType.DMA,
            pltpu.SemaphoreType.DMA,
            pltpu.SemaphoreType.DMA,
            pltpu.SemaphoreType.DMA,
        ],
    )


def _agg_call(row, col, w, table):
    return _built_agg()(row, col, w, table)


def _elu(t):
    return jnp.where(t > 0, t, jnp.exp(jnp.minimum(t, 0.0)) - 1.0)


def _prep_body(x_ref, w1_ref, deg_ref, dinv_ref, tab_ref):
    deg = deg_ref[:, 0:1] + 1.0
    dinv = lax.rsqrt(deg)
    dinv_ref[...] = dinv
    tab_ref[...] = jnp.dot(x_ref[...], w1_ref[...],
                           preferred_element_type=jnp.float32) * dinv


def _prep_call(x, W1, deg):
    return pl.pallas_call(
        _prep_body,
        out_shape=(jax.ShapeDtypeStruct((N, 1), jnp.float32),
                   jax.ShapeDtypeStruct((N, H), jnp.float32)),
    )(x, W1, deg)


def _agg_dense(acc_ref, tab_ref, dinv_ref, b_ref):
    t = acc_ref[...] + tab_ref[...]
    return _elu(t * dinv_ref[...] + b_ref[...][None, :])


def _mid_body(acc_ref, tab_ref, dinv_ref, b_ref, wn_ref, out_ref):
    t = _agg_dense(acc_ref, tab_ref, dinv_ref, b_ref)
    out_ref[...] = jnp.dot(t, wn_ref[...],
                           preferred_element_type=jnp.float32) * dinv_ref[...]


def _mid_call(acc, tab, dinv, b, Wn):
    return pl.pallas_call(
        _mid_body,
        out_shape=jax.ShapeDtypeStruct((N, H), jnp.float32),
    )(acc, tab, dinv, b, Wn)


def _final_body(acc_ref, tab_ref, dinv_ref, b_ref, batch_ref,
                fc1w_ref, fc1b_ref, fc2w_ref, fc2b_ref, out_ref):
    t = _agg_dense(acc_ref, tab_ref, dinv_ref, b_ref)
    gids = lax.broadcasted_iota(jnp.int32, (G, N), 0)
    onehot = (gids == batch_ref[...]).astype(jnp.float32)
    sums = jnp.dot(onehot, t, preferred_element_type=jnp.float32)
    cnt = jnp.sum(onehot, axis=1, keepdims=True)
    pooled = sums / jnp.maximum(cnt, 1.0)
    z = jnp.dot(pooled, fc1w_ref[...],
                preferred_element_type=jnp.float32) + fc1b_ref[...][None, :]
    z = _elu(z)
    o = jnp.dot(z, fc2w_ref[...],
                preferred_element_type=jnp.float32) + fc2b_ref[...][None, :]
    m = jnp.max(o, axis=1, keepdims=True)
    lse = m + jnp.log(jnp.sum(jnp.exp(o - m), axis=1, keepdims=True))
    out_ref[...] = o - lse


def _final_call(acc, tab, dinv, b4, batch2d, fc1_w, fc1_b, fc2_w, fc2_b):
    return pl.pallas_call(
        _final_body,
        out_shape=jax.ShapeDtypeStruct((G, C), jnp.float32),
    )(acc, tab, dinv, b4, batch2d, fc1_w, fc1_b, fc2_w, fc2_b)


def kernel(x, edge_index, edge_weight, batch,
           W1, b1, W2, b2, W3, b3, W4, b4,
           fc1_w, fc1_b, fc2_w, fc2_b):
    row = edge_index[0]
    col = edge_index[1]
    batch2d = batch.reshape(1, N)

    deg = _built_deg()(col, edge_weight)
    dinv, tab = _prep_call(x, W1, deg)
    for (bl, Wn) in ((b1, W2), (b2, W3), (b3, W4)):
        acc = _agg_call(row, col, edge_weight, tab)
        tab = _mid_call(acc, tab, dinv, bl, Wn)
    acc = _agg_call(row, col, edge_weight, tab)
    return _final_call(acc, tab, dinv, b4, batch2d,
                       fc1_w, fc1_b, fc2_w, fc2_b)
